# baseline (speedup 1.0000x reference)
import functools

import jax
import jax.numpy as jnp
from jax import lax
from jax.experimental import pallas as pl
from jax.experimental.pallas import tpu as pltpu
from jax.experimental.pallas import tpu_sc as plsc

POS_B = 4096
NEG_B = 16384
DIM = 128
NEG_RATIO = NEG_B // POS_B
N_RULES = 20
N_RULE_PAD = 32
MARGIN = 1.0
RULE_WEIGHT = 0.5

_NW = 32
_CH = 128
_NBUF = 4

_PB = 512
QH = POS_B // 2

ENT_A = 2 * POS_B + NEG_B
ENT_B = NEG_B
REL_A = POS_B + NEG_B // 2
REL_B = NEG_B // 2


_sc_mesh = plsc.VectorSubcoreMesh(core_axis_name="c", subcore_axis_name="s")


def _ring_pipeline(items, rows, gsem, ssem):
    n = len(items)
    gh = [None] * _NBUF
    sh = [None] * _NBUF
    issued = 0
    for k in range(n):
        while issued < min(n, k + _NBUF):
            b = issued % _NBUF
            if sh[b] is not None:
                sh[b].wait()
            tbl, isl, _, _ = items[issued]
            gh[b] = pltpu.async_copy(tbl.at[isl], rows.at[b], gsem[b])
            issued += 1
        b = k % _NBUF
        gh[b].wait()
        _, _, dst, off = items[k]
        sh[b] = pltpu.async_copy(rows.at[b], dst.at[pl.ds(off, _CH)], ssem[b])
    for b in range(min(_NBUF, n)):
        sh[b].wait()


def _make_sc_gather(ent_n, rel_n, with_rules):
    ent_pw = ent_n // _NW
    rel_pw = rel_n // _NW
    ent_ch = ent_pw // _CH
    rel_ch = rel_pw // _CH

    out_type = [
        jax.ShapeDtypeStruct((ent_n, DIM), jnp.float32),
        jax.ShapeDtypeStruct((rel_n, DIM), jnp.float32),
        jax.ShapeDtypeStruct((rel_n, DIM), jnp.float32),
    ]
    if with_rules:
        out_type += [
            jax.ShapeDtypeStruct((N_RULE_PAD, DIM), jnp.float32),
            jax.ShapeDtypeStruct((N_RULE_PAD, DIM), jnp.float32),
        ]
    scratch = [
        pltpu.VMEM((ent_pw,), jnp.int32),
        pltpu.VMEM((rel_pw,), jnp.int32),
        pltpu.VMEM((_NBUF, _CH, DIM), jnp.float32),
        pltpu.SemaphoreType.DMA,
    ] + [pltpu.SemaphoreType.DMA] * (2 * _NBUF)
    if with_rules:
        scratch += [
            pltpu.VMEM((N_RULE_PAD,), jnp.int32),
            pltpu.VMEM((N_RULE_PAD, DIM), jnp.float32),
        ]

    def body(*refs):
        ent_hbm, rel_hbm, norm_hbm, eidx_hbm, ridx_hbm = refs[:5]
        k = 5
        if with_rules:
            rulidx_hbm = refs[k]
            k += 1
        out_ent, out_rel, out_norm = refs[k:k + 3]
        k += 3
        if with_rules:
            out_dr, out_wr = refs[k:k + 2]
            k += 2
        idx_e, idx_r, rows, sem = refs[k:k + 4]
        k += 4
        ring_sems = refs[k:k + 2 * _NBUF]
        k += 2 * _NBUF
        if with_rules:
            idx_rul, rows_rul = refs[k:k + 2]

        wid = lax.axis_index("s") * 2 + lax.axis_index("c")
        gsem, ssem = ring_sems[:_NBUF], ring_sems[_NBUF:]

        pltpu.sync_copy(eidx_hbm.at[pl.ds(wid * ent_pw, ent_pw)], idx_e)
        pltpu.sync_copy(ridx_hbm.at[pl.ds(wid * rel_pw, rel_pw)], idx_r)

        items = []
        for c in range(ent_ch):
            items.append((ent_hbm, idx_e.at[pl.ds(c * _CH, _CH)],
                          out_ent, wid * ent_pw + c * _CH))
        for c in range(rel_ch):
            isl = idx_r.at[pl.ds(c * _CH, _CH)]
            off = wid * rel_pw + c * _CH
            items.append((rel_hbm, isl, out_rel, off))
            items.append((norm_hbm, isl, out_norm, off))
        _ring_pipeline(items, rows, gsem, ssem)

        if with_rules:
            @pl.when(wid == 0)
            def _():
                pltpu.sync_copy(rulidx_hbm, idx_rul)
                pltpu.async_copy(rel_hbm.at[idx_rul], rows_rul, sem).wait()
                pltpu.sync_copy(rows_rul, out_dr)
                pltpu.async_copy(norm_hbm.at[idx_rul], rows_rul, sem).wait()
                pltpu.sync_copy(rows_rul, out_wr)

    return pl.kernel(body, mesh=_sc_mesh, out_type=out_type,
                     scratch_types=scratch)


_sc_gather_a = _make_sc_gather(ENT_A, REL_A, True)
_sc_gather_b = _make_sc_gather(ENT_B, REL_B, False)


def _normw(w):
    return w / (jnp.sqrt(jnp.sum(w * w, axis=-1, keepdims=True)) + 1e-9)


def _score_u(u, d, w):
    wn = _normw(w)
    al = jnp.sum(wn * u, axis=-1, keepdims=True)
    v = u - al * wn + d
    return -jnp.sqrt(jnp.sum(v * v, axis=-1, keepdims=True) + 1e-12)


def _tc1_body(hp, tp, hn0, hn1, tn0, tn1, dp, dn0, dn1, wp, wn0, wn1,
              dr, wr, posr, r1b, confb, out, ps_out):
    i = pl.program_id(0)

    up = hp[...] - tp[...]
    ps = _score_u(up, dp[...], wp[...])
    ps_out[...] = ps

    basic = jnp.float32(0.0)
    for hn, tn, dn, wn in ((hn0, tn0, dn0, wn0), (hn1, tn1, dn1, wn1)):
        ns = _score_u(hn[...] - tn[...], dn[...], wn[...])
        basic = basic + jnp.sum(jax.nn.relu(MARGIN - ps + ns))

    drv = dr[...]
    wrv = _normw(wr[...])
    dnum = (((1,), (1,)), ((), ()))
    alr = lax.dot_general(up, wrv, dnum, preferred_element_type=jnp.float32)
    ber = lax.dot_general(up, drv, dnum, preferred_element_type=jnp.float32)
    ones = jnp.ones((1, DIM), jnp.float32)
    ddr = lax.dot_general(ones, drv * drv, dnum,
                          preferred_element_type=jnp.float32)
    wdr = lax.dot_general(ones, wrv * drv, dnum,
                          preferred_element_type=jnp.float32)
    nu = jnp.sum(up * up, axis=-1, keepdims=True)
    dist2 = nu - alr * alr + ddr + 2.0 * ber - 2.0 * alr * wdr
    rsc = -jnp.sqrt(jnp.maximum(dist2, 0.0) + 1e-12)
    mask = posr[...] == r1b[0:1, :]
    rulep = -jnp.sum(jnp.where(mask, confb[0:1, :] * rsc, 0.0))

    part = basic * (1.0 / NEG_B) + RULE_WEIGHT * rulep

    @pl.when(i == 0)
    def _():
        out[...] = jnp.zeros_like(out)

    out[...] += part


def _tc2_body(hn2, hn3, tn2, tn3, dn2, dn3, wn2, wn3, ps_in, out):
    i = pl.program_id(0)
    ps = ps_in[...]
    basic = jnp.float32(0.0)
    for hn, tn, dn, wn in ((hn2, tn2, dn2, wn2), (hn3, tn3, dn3, wn3)):
        ns = _score_u(hn[...] - tn[...], dn[...], wn[...])
        basic = basic + jnp.sum(jax.nn.relu(MARGIN - ps + ns))

    @pl.when(i == 0)
    def _():
        out[...] = jnp.zeros_like(out)

    out[...] += basic * (1.0 / NEG_B)


def _tc1_call(ent_a, rel_a, norm_a, dr_rows, wr_rows, posr, r1b, confb):
    g = POS_B // _PB
    ebs = lambda f: pl.BlockSpec((_PB, DIM), f)
    specs = [
        ebs(lambda i: (i, 0)),
        ebs(lambda i: (i + 8, 0)),
        ebs(lambda i: (16 + i, 0)),
        ebs(lambda i: (24 + i, 0)),
        ebs(lambda i: (32 + i, 0)),
        ebs(lambda i: (40 + i, 0)),
        ebs(lambda i: (i, 0)),
        ebs(lambda i: (8 + i, 0)),
        ebs(lambda i: (16 + i, 0)),
        ebs(lambda i: (i, 0)),
        ebs(lambda i: (8 + i, 0)),
        ebs(lambda i: (16 + i, 0)),
        pl.BlockSpec((N_RULE_PAD, DIM), lambda i: (0, 0)),
        pl.BlockSpec((N_RULE_PAD, DIM), lambda i: (0, 0)),
        pl.BlockSpec((_PB, 1), lambda i: (i, 0)),
        pl.BlockSpec((8, N_RULE_PAD), lambda i: (0, 0)),
        pl.BlockSpec((8, N_RULE_PAD), lambda i: (0, 0)),
    ]
    return pl.pallas_call(
        _tc1_body,
        grid=(g,),
        in_specs=specs,
        out_specs=[pl.BlockSpec((1, 1), lambda i: (0, 0)),
                   pl.BlockSpec((_PB, 1), lambda i: (i, 0))],
        out_shape=[jax.ShapeDtypeStruct((1, 1), jnp.float32),
                   jax.ShapeDtypeStruct((POS_B, 1), jnp.float32)],
    )(ent_a, ent_a, ent_a, ent_a, ent_a, ent_a,
      rel_a, rel_a, rel_a, norm_a, norm_a, norm_a,
      dr_rows, wr_rows, posr, r1b, confb)


def _tc2_call(ent_b, rel_b, norm_b, ps):
    g = POS_B // _PB
    ebs = lambda f: pl.BlockSpec((_PB, DIM), f)
    specs = [
        ebs(lambda i: (i, 0)),
        ebs(lambda i: (8 + i, 0)),
        ebs(lambda i: (16 + i, 0)),
        ebs(lambda i: (24 + i, 0)),
        ebs(lambda i: (i, 0)),
        ebs(lambda i: (8 + i, 0)),
        ebs(lambda i: (i, 0)),
        ebs(lambda i: (8 + i, 0)),
        pl.BlockSpec((_PB, 1), lambda i: (i, 0)),
    ]
    return pl.pallas_call(
        _tc2_body,
        grid=(g,),
        in_specs=specs,
        out_specs=pl.BlockSpec((1, 1), lambda i: (0, 0)),
        out_shape=jax.ShapeDtypeStruct((1, 1), jnp.float32),
    )(ent_b, ent_b, ent_b, ent_b, rel_b, rel_b, norm_b, norm_b, ps)


def kernel(pos_triples, neg_triples, ent_emb, rel_emb, norm_vec,
           rule_r1, rule_r2, rule_conf):
    ph, pr, pt = pos_triples[:, 0], pos_triples[:, 1], pos_triples[:, 2]
    nh, nr, nt = neg_triples[:, 0], neg_triples[:, 1], neg_triples[:, 2]

    qmaj = lambda x: x.reshape(POS_B, NEG_RATIO).T.reshape(-1)
    nhq, ntq, nrq = qmaj(nh), qmaj(nt), qmaj(nr)

    half = NEG_B // 2
    eidx_a = jnp.concatenate([ph, pt, nhq[:half], ntq[:half]])
    eidx_b = jnp.concatenate([nhq[half:], ntq[half:]])
    ridx_a = jnp.concatenate([pr, nrq[:half]])
    ridx_b = nrq[half:]
    rulidx = jnp.concatenate(
        [rule_r2, jnp.zeros((N_RULE_PAD - N_RULES,), jnp.int32)])

    ent_a, rel_a, norm_a, dr_rows, wr_rows = _sc_gather_a(
        ent_emb, rel_emb, norm_vec, eidx_a, ridx_a, rulidx)
    ent_b, rel_b, norm_b = _sc_gather_b(
        ent_emb, rel_emb, norm_vec, eidx_b, ridx_b)

    posr = pr.reshape(POS_B, 1)
    pad_i = jnp.full((N_RULE_PAD - N_RULES,), -1, jnp.int32)
    r1b = jnp.broadcast_to(
        jnp.concatenate([rule_r1, pad_i])[None, :], (8, N_RULE_PAD))
    confb = jnp.broadcast_to(
        jnp.concatenate([rule_conf, jnp.zeros((N_RULE_PAD - N_RULES,),
                                              jnp.float32)])[None, :],
        (8, N_RULE_PAD))

    loss1, ps = _tc1_call(ent_a, rel_a, norm_a, dr_rows, wr_rows,
                          posr, r1b, confb)
    loss2 = _tc2_call(ent_b, rel_b, norm_b, ps)
    return (loss1 + loss2).reshape(())
